# baseline (speedup 1.0000x reference)
import functools

import jax
import jax.numpy as jnp
from jax import lax
from jax.experimental import pallas as pl
from jax.experimental.pallas import tpu as pltpu
from jax.experimental.pallas import tpu_sc as plsc

_B, _C, _H, _W = 8, 2, 512, 512
_N = _B * _H * _W
_RB = 512
_GRID = (_B, _H // _RB)


def _ce_block(pred_ref, gt_ref):
    p0 = pred_ref[0, 0]
    p1 = pred_ref[0, 1]
    g = gt_ref[0, 0]
    d = p0 - p1
    sp = jnp.log(1.0 + jnp.exp(-jnp.abs(d)))
    m_minus = jnp.where(g == 0, jnp.maximum(-d, 0.0), jnp.maximum(d, 0.0))
    return sp + m_minus


def _sums_body(pred_ref, gt_ref, wt_ref, acc_ref):
    i = pl.program_id(0)
    j = pl.program_id(1)

    @pl.when((i == 0) & (j == 0))
    def _():
        acc_ref[0] = 0.0
        acc_ref[1] = 0.0
        acc_ref[2] = 0.0

    ce = _ce_block(pred_ref, gt_ref)
    w = wt_ref[0, 0]
    acc_ref[0] += jnp.sum(ce)
    acc_ref[1] += jnp.sum(w * ce)
    acc_ref[2] += jnp.sum(jnp.where(gt_ref[0, 0] != 0, 1.0, 0.0))


def _temp_body(pred_ref, gt_ref, wt_ref, out_ref):
    ce = _ce_block(pred_ref, gt_ref)
    out_ref[0, 0] = jnp.where(wt_ref[0, 0] != 0.0, 0.0, ce)


def _count_body(mid_ref, temp_ref, cnt_ref):
    i = pl.program_id(0)
    j = pl.program_id(1)

    @pl.when((i == 0) & (j == 0))
    def _():
        cnt_ref[0] = 0.0

    cnt_ref[0] += jnp.sum(jnp.where(temp_ref[0, 0] >= mid_ref[0], 1.0, 0.0))


def _tail_body(thr_ref, temp_ref, out_ref):
    i = pl.program_id(0)
    j = pl.program_id(1)

    @pl.when((i == 0) & (j == 0))
    def _():
        out_ref[0] = 0.0
        out_ref[1] = 0.0

    t = temp_ref[0, 0]
    gt_mask = t > thr_ref[0]
    out_ref[0] += jnp.sum(jnp.where(gt_mask, t, 0.0))
    out_ref[1] += jnp.sum(jnp.where(gt_mask, 1.0, 0.0))


_pred_spec = pl.BlockSpec((1, _C, _RB, _W), lambda i, j: (i, 0, j, 0))
_map_spec = pl.BlockSpec((1, 1, _RB, _W), lambda i, j: (i, 0, j, 0))
_smem_scalar = pl.BlockSpec(memory_space=pltpu.SMEM)


def _topk_sum(pred, gt, wt, k_f32):
    temp = pl.pallas_call(
        _temp_body,
        grid=_GRID,
        in_specs=[_pred_spec, _map_spec, _map_spec],
        out_specs=_map_spec,
        out_shape=jax.ShapeDtypeStruct((_B, 1, _H, _W), jnp.float32),
    )(pred, gt, wt)

    count_call = pl.pallas_call(
        _count_body,
        grid=_GRID,
        in_specs=[_smem_scalar, _map_spec],
        out_specs=_smem_scalar,
        out_shape=jax.ShapeDtypeStruct((1,), jnp.float32),
    )

    def cond(c):
        lo, hi = c
        return hi - lo > 1

    def body(c):
        lo, hi = c
        mid = lo + (hi - lo) // 2
        midf = lax.bitcast_convert_type(mid, jnp.float32)
        cnt = count_call(midf.reshape(1), temp)[0]
        ge = cnt >= k_f32
        return (jnp.where(ge, mid, lo), jnp.where(ge, hi, mid))

    lo0 = jnp.int32(0)
    hi0 = jnp.int32(0x7F800000)
    lo, _ = lax.while_loop(cond, body, (lo0, hi0))
    thr = lax.bitcast_convert_type(lo, jnp.float32)

    tail = pl.pallas_call(
        _tail_body,
        grid=_GRID,
        in_specs=[_smem_scalar, _map_spec],
        out_specs=_smem_scalar,
        out_shape=jax.ShapeDtypeStruct((2,), jnp.float32),
    )(thr.reshape(1), temp)
    sum_gt, cnt_gt = tail[0], tail[1]
    return sum_gt + (k_f32 - cnt_gt) * thr


_SC_CH = 16
_SC_NW = 32


def _sc_sums(pred, gt, wt, row_start, n_rows):
    rpw = n_rows // _SC_NW
    n_chunks = rpw // _SC_CH
    mesh = plsc.VectorSubcoreMesh(core_axis_name="c", subcore_axis_name="s")
    vm_f = pltpu.VMEM((_SC_CH, _W), jnp.float32)
    vm_i = pltpu.VMEM((_SC_CH, _W), jnp.int32)

    @functools.partial(
        pl.kernel,
        mesh=mesh,
        out_type=jax.ShapeDtypeStruct((_SC_NW * 48,), jnp.float32),
        scratch_types=[
            vm_f, vm_f, vm_i, vm_f,
            vm_f, vm_f, vm_i, vm_f,
            pltpu.VMEM((48,), jnp.float32),
            pltpu.SemaphoreType.DMA,
            pltpu.SemaphoreType.DMA,
        ],
    )
    def k(pred_h, gt_h, wt_h, out_h,
          p0a, p1a, ga, wa, p0b, p1b, gb, wb, stage, sem_a, sem_b):
        wid = lax.axis_index("s") * 2 + lax.axis_index("c")
        row_base = row_start + wid * rpw
        bufs = ((p0a, p1a, ga, wa, sem_a), (p0b, p1b, gb, wb, sem_b))

        def issue(t):
            p0v, p1v, gv, wv, sem = bufs[t % 2]
            gr = row_base + t * _SC_CH
            b = gr // _H
            r0 = gr % _H
            return [
                pltpu.async_copy(pred_h.at[b, 0, pl.ds(r0, _SC_CH), :], p0v, sem),
                pltpu.async_copy(pred_h.at[b, 1, pl.ds(r0, _SC_CH), :], p1v, sem),
                pltpu.async_copy(gt_h.at[b, 0, pl.ds(r0, _SC_CH), :], gv, sem),
                pltpu.async_copy(wt_h.at[b, 0, pl.ds(r0, _SC_CH), :], wv, sem),
            ]

        def chunk_sums(t, accs):
            p0v, p1v, gv, wv, _ = bufs[t % 2]

            def body(idx, accs):
                sa, sw, ar = accs
                base = idx * 8
                i = base // 32
                cb = base % 32
                for u in range(8):
                    sl = (i, pl.ds((cb + u) * 16, 16))
                    p0 = p0v[sl]
                    p1 = p1v[sl]
                    g = gv[sl]
                    w = wv[sl]
                    d = p0 - p1
                    u_ = jnp.exp(-jnp.abs(d))
                    z = u_ / (2.0 + u_)
                    z2 = z * z
                    poly = 1.0 / 5.0 + z2 * (1.0 / 7.0)
                    poly = 1.0 / 3.0 + z2 * poly
                    sp = 2.0 * z * (1.0 + z2 * poly)
                    ce = sp + jnp.where(g == 0, jnp.maximum(-d, 0.0),
                                        jnp.maximum(d, 0.0))
                    sa = sa + ce
                    sw = sw + w * ce
                    ar = ar + jnp.where(g != 0, 1.0, 0.0)
                return (sa, sw, ar)

            return lax.fori_loop(0, _SC_CH * (_W // 16) // 8, body, accs)

        zero = jnp.zeros((16,), jnp.float32)
        accs = (zero, zero, zero)
        handles = {0: issue(0)}
        if n_chunks > 1:
            handles[1] = issue(1)
        for t in range(n_chunks):
            for h in handles.pop(t):
                h.wait()
            accs = chunk_sums(t, accs)
            if t + 2 < n_chunks:
                handles[t + 2] = issue(t + 2)

        sa, sw, ar = accs
        stage[pl.ds(0, 16)] = sa
        stage[pl.ds(16, 16)] = sw
        stage[pl.ds(32, 16)] = ar
        pltpu.sync_copy(stage, out_h.at[pl.ds(wid * 48, 48)])

    return k(pred, gt, wt)


_B_TC = 7


def kernel(pixel_pred, pixel_gt, pixel_weight):
    parts = _sc_sums(pixel_pred, pixel_gt, pixel_weight,
                     _B_TC * _H, (_B - _B_TC) * _H)
    tc_acc = pl.pallas_call(
        _sums_body,
        grid=(_B_TC, 1),
        in_specs=[_pred_spec, _map_spec, _map_spec],
        out_specs=_smem_scalar,
        out_shape=jax.ShapeDtypeStruct((3,), jnp.float32),
    )(pixel_pred, pixel_gt, pixel_weight)
    pm = parts.reshape(_SC_NW, 3, 16)
    sums = tc_acc + jnp.sum(pm, axis=(0, 2))
    s_all, s_w, area = sums[0], sums[1], sums[2]

    k_f32 = jnp.minimum(3.0 * area, float(_N))
    total = lax.cond(
        3.0 * area >= float(_N),
        lambda: s_all,
        lambda: s_w + _topk_sum(pixel_pred, pixel_gt, pixel_weight, k_f32),
    )
    return total / (4.0 * area)


# speedup vs baseline: 1.0112x; 1.0112x over previous
import functools

import jax
import jax.numpy as jnp
from jax import lax
from jax.experimental import pallas as pl
from jax.experimental.pallas import tpu as pltpu
from jax.experimental.pallas import tpu_sc as plsc

_B, _C, _H, _W = 8, 2, 512, 512
_N = _B * _H * _W
_RB = 512
_GRID = (_B, _H // _RB)


def _ce_block(pred_ref, gt_ref):
    p0 = pred_ref[0, 0]
    p1 = pred_ref[0, 1]
    g = gt_ref[0, 0]
    d = p0 - p1
    sp = jnp.log(1.0 + jnp.exp(-jnp.abs(d)))
    m_minus = jnp.where(g == 0, jnp.maximum(-d, 0.0), jnp.maximum(d, 0.0))
    return sp + m_minus


def _sums_body(pred_ref, gt_ref, wt_ref, acc_ref):
    i = pl.program_id(0)
    j = pl.program_id(1)

    @pl.when((i == 0) & (j == 0))
    def _():
        acc_ref[0] = 0.0
        acc_ref[1] = 0.0
        acc_ref[2] = 0.0

    ce = _ce_block(pred_ref, gt_ref)
    w = wt_ref[0, 0]
    acc_ref[0] += jnp.sum(ce)
    acc_ref[1] += jnp.sum(w * ce)
    acc_ref[2] += jnp.sum(jnp.where(gt_ref[0, 0] != 0, 1.0, 0.0))


def _temp_body(pred_ref, gt_ref, wt_ref, out_ref):
    ce = _ce_block(pred_ref, gt_ref)
    out_ref[0, 0] = jnp.where(wt_ref[0, 0] != 0.0, 0.0, ce)


def _count_body(mid_ref, temp_ref, cnt_ref):
    i = pl.program_id(0)
    j = pl.program_id(1)

    @pl.when((i == 0) & (j == 0))
    def _():
        cnt_ref[0] = 0.0

    cnt_ref[0] += jnp.sum(jnp.where(temp_ref[0, 0] >= mid_ref[0], 1.0, 0.0))


def _tail_body(thr_ref, temp_ref, out_ref):
    i = pl.program_id(0)
    j = pl.program_id(1)

    @pl.when((i == 0) & (j == 0))
    def _():
        out_ref[0] = 0.0
        out_ref[1] = 0.0

    t = temp_ref[0, 0]
    gt_mask = t > thr_ref[0]
    out_ref[0] += jnp.sum(jnp.where(gt_mask, t, 0.0))
    out_ref[1] += jnp.sum(jnp.where(gt_mask, 1.0, 0.0))


_pred_spec = pl.BlockSpec((1, _C, _RB, _W), lambda i, j: (i, 0, j, 0))
_map_spec = pl.BlockSpec((1, 1, _RB, _W), lambda i, j: (i, 0, j, 0))
_smem_scalar = pl.BlockSpec(memory_space=pltpu.SMEM)


def _topk_sum(pred, gt, wt, k_f32):
    temp = pl.pallas_call(
        _temp_body,
        grid=_GRID,
        in_specs=[_pred_spec, _map_spec, _map_spec],
        out_specs=_map_spec,
        out_shape=jax.ShapeDtypeStruct((_B, 1, _H, _W), jnp.float32),
    )(pred, gt, wt)

    count_call = pl.pallas_call(
        _count_body,
        grid=_GRID,
        in_specs=[_smem_scalar, _map_spec],
        out_specs=_smem_scalar,
        out_shape=jax.ShapeDtypeStruct((1,), jnp.float32),
    )

    def cond(c):
        lo, hi = c
        return hi - lo > 1

    def body(c):
        lo, hi = c
        mid = lo + (hi - lo) // 2
        midf = lax.bitcast_convert_type(mid, jnp.float32)
        cnt = count_call(midf.reshape(1), temp)[0]
        ge = cnt >= k_f32
        return (jnp.where(ge, mid, lo), jnp.where(ge, hi, mid))

    lo0 = jnp.int32(0)
    hi0 = jnp.int32(0x7F800000)
    lo, _ = lax.while_loop(cond, body, (lo0, hi0))
    thr = lax.bitcast_convert_type(lo, jnp.float32)

    tail = pl.pallas_call(
        _tail_body,
        grid=_GRID,
        in_specs=[_smem_scalar, _map_spec],
        out_specs=_smem_scalar,
        out_shape=jax.ShapeDtypeStruct((2,), jnp.float32),
    )(thr.reshape(1), temp)
    sum_gt, cnt_gt = tail[0], tail[1]
    return sum_gt + (k_f32 - cnt_gt) * thr


_SC_CH = 8
_SC_NW = 32


def _sc_sums(pred, gt, wt, row_start, n_rows):
    rpw = n_rows // _SC_NW
    n_chunks = rpw // _SC_CH
    mesh = plsc.VectorSubcoreMesh(core_axis_name="c", subcore_axis_name="s")
    vm_f = pltpu.VMEM((_SC_CH, _W), jnp.float32)
    vm_i = pltpu.VMEM((_SC_CH, _W), jnp.int32)

    @functools.partial(
        pl.kernel,
        mesh=mesh,
        out_type=jax.ShapeDtypeStruct((_SC_NW * 48,), jnp.float32),
        scratch_types=[
            vm_f, vm_f, vm_i, vm_f,
            vm_f, vm_f, vm_i, vm_f,
            pltpu.VMEM((48,), jnp.float32),
            pltpu.SemaphoreType.DMA,
            pltpu.SemaphoreType.DMA,
        ],
    )
    def k(pred_h, gt_h, wt_h, out_h,
          p0a, p1a, ga, wa, p0b, p1b, gb, wb, stage, sem_a, sem_b):
        wid = lax.axis_index("s") * 2 + lax.axis_index("c")
        row_base = row_start + wid * rpw
        bufs = ((p0a, p1a, ga, wa, sem_a), (p0b, p1b, gb, wb, sem_b))

        def issue(t):
            p0v, p1v, gv, wv, sem = bufs[t % 2]
            gr = row_base + t * _SC_CH
            b = gr // _H
            r0 = gr % _H
            return [
                pltpu.async_copy(pred_h.at[b, 0, pl.ds(r0, _SC_CH), :], p0v, sem),
                pltpu.async_copy(pred_h.at[b, 1, pl.ds(r0, _SC_CH), :], p1v, sem),
                pltpu.async_copy(gt_h.at[b, 0, pl.ds(r0, _SC_CH), :], gv, sem),
                pltpu.async_copy(wt_h.at[b, 0, pl.ds(r0, _SC_CH), :], wv, sem),
            ]

        def chunk_sums(t, accs):
            p0v, p1v, gv, wv, _ = bufs[t % 2]

            def body(idx, accs):
                sa, sw, ar = accs
                base = idx * 8
                i = base // 32
                cb = base % 32
                for u in range(8):
                    sl = (i, pl.ds((cb + u) * 16, 16))
                    p0 = p0v[sl]
                    p1 = p1v[sl]
                    g = gv[sl]
                    w = wv[sl]
                    d = p0 - p1
                    u_ = jnp.exp(-jnp.abs(d))
                    z = u_ / (2.0 + u_)
                    z2 = z * z
                    poly = 1.0 / 5.0 + z2 * (1.0 / 7.0)
                    poly = 1.0 / 3.0 + z2 * poly
                    sp = 2.0 * z * (1.0 + z2 * poly)
                    ce = sp + jnp.where(g == 0, jnp.maximum(-d, 0.0),
                                        jnp.maximum(d, 0.0))
                    sa = sa + ce
                    sw = sw + w * ce
                    ar = ar + jnp.where(g != 0, 1.0, 0.0)
                return (sa, sw, ar)

            return lax.fori_loop(0, _SC_CH * (_W // 16) // 8, body, accs)

        zero = jnp.zeros((16,), jnp.float32)
        accs = (zero, zero, zero)
        handles = {0: issue(0)}
        if n_chunks > 1:
            handles[1] = issue(1)
        for t in range(n_chunks):
            for h in handles.pop(t):
                h.wait()
            accs = chunk_sums(t, accs)
            if t + 2 < n_chunks:
                handles[t + 2] = issue(t + 2)

        sa, sw, ar = accs
        stage[pl.ds(0, 16)] = sa
        stage[pl.ds(16, 16)] = sw
        stage[pl.ds(32, 16)] = ar
        pltpu.sync_copy(stage, out_h.at[pl.ds(wid * 48, 48)])

    return k(pred, gt, wt)


_B_TC = 7


def kernel(pixel_pred, pixel_gt, pixel_weight):
    parts = _sc_sums(pixel_pred, pixel_gt, pixel_weight,
                     _B_TC * _H, (_B - _B_TC) * _H)
    tc_acc = pl.pallas_call(
        _sums_body,
        grid=(_B_TC, 1),
        in_specs=[_pred_spec, _map_spec, _map_spec],
        out_specs=_smem_scalar,
        out_shape=jax.ShapeDtypeStruct((3,), jnp.float32),
    )(pixel_pred, pixel_gt, pixel_weight)
    pm = parts.reshape(_SC_NW, 3, 16)
    sums = tc_acc + jnp.sum(pm, axis=(0, 2))
    s_all, s_w, area = sums[0], sums[1], sums[2]

    k_f32 = jnp.minimum(3.0 * area, float(_N))
    total = lax.cond(
        3.0 * area >= float(_N),
        lambda: s_all,
        lambda: s_w + _topk_sum(pixel_pred, pixel_gt, pixel_weight, k_f32),
    )
    return total / (4.0 * area)
